# baseline (speedup 1.0000x reference)
import functools

import jax
import jax.numpy as jnp
from jax import lax
from jax.experimental import pallas as pl
from jax.experimental.pallas import tpu as pltpu
from jax.experimental.pallas import tpu_sc as plsc

N_NODES = 10000
N_EDGES = 320000
NFEAT = 128

NC = 2
NS = 16
NW = NC * NS
EPT = N_EDGES // NW
CHUNK = 384
NFULL = 27
EPT_PAD = NFULL * CHUNK
N_PAD = 10008
ROWS_PER_TILE = 640

_mesh = plsc.VectorSubcoreMesh(core_axis_name="c", subcore_axis_name="s")


@functools.partial(
    pl.kernel,
    mesh=_mesh,
    out_type=jax.ShapeDtypeStruct((NW, N_NODES), jnp.float32),
    scratch_types=[
        pltpu.VMEM((N_NODES,), jnp.float32),
        pltpu.VMEM((EPT,), jnp.int32),
    ],
    compiler_params=pltpu.CompilerParams(needs_layout_passes=False),
)
def _deg_kernel(dst_hbm, out_hbm, hist, dstv):
    c = lax.axis_index("c")
    s = lax.axis_index("s")
    wid = c * NS + s

    def zero(i, carry):
        hist[pl.ds(i * 16, 16)] = jnp.zeros((16,), jnp.float32)
        return carry

    lax.fori_loop(0, N_NODES // 16, zero, 0)

    pltpu.sync_copy(dst_hbm.at[pl.ds(wid * EPT, EPT)], dstv)
    ones = jnp.ones((16,), jnp.float32)

    def step(i, carry):
        idx = dstv[pl.ds(i * 16, 16)]
        plsc.addupdate_scatter(hist, [idx], ones)
        return carry

    lax.fori_loop(0, EPT // 16, step, 0)
    pltpu.sync_copy(hist, out_hbm.at[wid])


@functools.partial(
    pl.kernel,
    mesh=_mesh,
    out_type=jax.ShapeDtypeStruct((NC, N_PAD, NFEAT), jnp.float32),
    scratch_types=[
        pltpu.VMEM_SHARED((N_PAD, NFEAT), jnp.float32),
        pltpu.VMEM((CHUNK,), jnp.int32),
        pltpu.VMEM((CHUNK,), jnp.int32),
        pltpu.VMEM((CHUNK, NFEAT), jnp.float32),
        pltpu.SemaphoreType.DMA,
    ],
    compiler_params=pltpu.CompilerParams(needs_layout_passes=False),
)
def _scatter_kernel(h_hbm, src_hbm, dst_hbm, zeros_hbm, out_hbm,
                    acc, srcv, dstv, rows, gsem):
    c = lax.axis_index("c")
    s = lax.axis_index("s")

    @pl.when(s < NS - 1)
    def _():
        pltpu.sync_copy(zeros_hbm.at[pl.ds(s * ROWS_PER_TILE, ROWS_PER_TILE)],
                        acc.at[pl.ds(s * ROWS_PER_TILE, ROWS_PER_TILE)])

    @pl.when(s == NS - 1)
    def _():
        last = (NS - 1) * ROWS_PER_TILE
        pltpu.sync_copy(zeros_hbm.at[pl.ds(last, N_PAD - last)],
                        acc.at[pl.ds(last, N_PAD - last)])

    plsc.subcore_barrier()

    wid = c * NS + s
    ebase = wid * EPT_PAD

    def step(i, carry):
        base = ebase + i * CHUNK
        pltpu.sync_copy(src_hbm.at[pl.ds(base, CHUNK)], srcv)
        pltpu.sync_copy(dst_hbm.at[pl.ds(base, CHUNK)], dstv)
        pltpu.async_copy(h_hbm.at[srcv], rows, gsem).wait()
        pltpu.sync_copy(rows, acc.at[dstv], add=True)
        return carry

    lax.fori_loop(0, NFULL, step, 0)
    plsc.subcore_barrier()

    @pl.when(s < NS - 1)
    def _():
        pltpu.sync_copy(acc.at[pl.ds(s * ROWS_PER_TILE, ROWS_PER_TILE)],
                        out_hbm.at[c, pl.ds(s * ROWS_PER_TILE, ROWS_PER_TILE)])

    @pl.when(s == NS - 1)
    def _():
        last = (NS - 1) * ROWS_PER_TILE
        pltpu.sync_copy(acc.at[pl.ds(last, N_PAD - last)],
                        out_hbm.at[c, pl.ds(last, N_PAD - last)])


_RB = 1000
_GRID = N_NODES // _RB


def _mm1_body(x_ref, w_ref, o_ref):
    o_ref[...] = jnp.dot(x_ref[...], w_ref[...],
                         preferred_element_type=jnp.float32)


def _mm1(x, w):
    return pl.pallas_call(
        _mm1_body,
        grid=(_GRID,),
        in_specs=[
            pl.BlockSpec((_RB, NFEAT), lambda i: (i, 0)),
            pl.BlockSpec((NFEAT, NFEAT), lambda i: (0, 0)),
        ],
        out_specs=pl.BlockSpec((_RB, NFEAT), lambda i: (i, 0)),
        out_shape=jax.ShapeDtypeStruct((N_NODES, NFEAT), jnp.float32),
    )(x, w)


def _dinv_body(hist_ref, dinv_ref):
    deg = 1.0 + jnp.sum(hist_ref[...], axis=0)
    dinv_ref[...] = lax.rsqrt(deg)[:, None]


def _dinv(hist):
    return pl.pallas_call(
        _dinv_body,
        grid=(1,),
        in_specs=[pl.BlockSpec((NW, N_NODES), lambda i: (0, 0))],
        out_specs=pl.BlockSpec((N_NODES, 1), lambda i: (0, 0)),
        out_shape=jax.ShapeDtypeStruct((N_NODES, 1), jnp.float32),
    )(hist)


def _scale_body(h1_ref, dinv_ref, h1p_ref):
    h1p_ref[...] = h1_ref[...] * dinv_ref[...]


def _scale(h1, dinv):
    return pl.pallas_call(
        _scale_body,
        grid=(_GRID,),
        in_specs=[
            pl.BlockSpec((_RB, NFEAT), lambda i: (i, 0)),
            pl.BlockSpec((_RB, 1), lambda i: (i, 0)),
        ],
        out_specs=pl.BlockSpec((_RB, NFEAT), lambda i: (i, 0)),
        out_shape=jax.ShapeDtypeStruct((N_NODES, NFEAT), jnp.float32),
    )(h1, dinv)


def _mid_body(p_ref, h1p_ref, dinv_ref, b1_ref, w2_ref, h2p_ref):
    psum = p_ref[0] + p_ref[1]
    u = (psum + h1p_ref[...]) * dinv_ref[...] + b1_ref[...]
    u = jnp.maximum(u, 0.0)
    h2 = jnp.dot(u, w2_ref[...], preferred_element_type=jnp.float32)
    h2p_ref[...] = h2 * dinv_ref[...]


def _mid(p1, h1p, dinv, b1, w2):
    return pl.pallas_call(
        _mid_body,
        grid=(_GRID,),
        in_specs=[
            pl.BlockSpec((NC, _RB, NFEAT), lambda i: (0, i, 0)),
            pl.BlockSpec((_RB, NFEAT), lambda i: (i, 0)),
            pl.BlockSpec((_RB, 1), lambda i: (i, 0)),
            pl.BlockSpec((1, NFEAT), lambda i: (0, 0)),
            pl.BlockSpec((NFEAT, NFEAT), lambda i: (0, 0)),
        ],
        out_specs=pl.BlockSpec((_RB, NFEAT), lambda i: (i, 0)),
        out_shape=jax.ShapeDtypeStruct((N_NODES, NFEAT), jnp.float32),
    )(p1, h1p, dinv, b1, w2)


def _fin_body(p_ref, h2p_ref, dinv_ref, b2_ref, wfc_ref, bfc_ref, o_ref):
    v = (p_ref[0] + p_ref[1] + h2p_ref[...]) * dinv_ref[...] + b2_ref[...]
    o_ref[...] = jnp.dot(v, wfc_ref[...],
                         preferred_element_type=jnp.float32) + bfc_ref[0, 0]


def _fin(p2, h2p, dinv, b2, wfc, bfc):
    return pl.pallas_call(
        _fin_body,
        grid=(_GRID,),
        in_specs=[
            pl.BlockSpec((NC, _RB, NFEAT), lambda i: (0, i, 0)),
            pl.BlockSpec((_RB, NFEAT), lambda i: (i, 0)),
            pl.BlockSpec((_RB, 1), lambda i: (i, 0)),
            pl.BlockSpec((1, NFEAT), lambda i: (0, 0)),
            pl.BlockSpec((NFEAT, 1), lambda i: (0, 0)),
            pl.BlockSpec((1, 1), lambda i: (0, 0)),
        ],
        out_specs=pl.BlockSpec((_RB, 1), lambda i: (i, 0)),
        out_shape=jax.ShapeDtypeStruct((N_NODES, 1), jnp.float32),
    )(p2, h2p, dinv, b2, wfc, bfc)


def _pad_edges(v):
    pad = jnp.full((NW, EPT_PAD - EPT), N_NODES, jnp.int32)
    out = jnp.concatenate([v.reshape(NW, EPT), pad], axis=1)
    return out.reshape(-1)


def _pad_rows(h):
    return jnp.concatenate(
        [h, jnp.zeros((N_PAD - N_NODES, NFEAT), jnp.float32)], axis=0)


def kernel(x, edge_index, W1, b1, W2, b2, Wfc, bfc):
    ei = edge_index.astype(jnp.int32)
    src = _pad_edges(ei[0])
    dst = _pad_edges(ei[1])
    zeros = jnp.zeros((N_PAD, NFEAT), jnp.float32)

    hist = _deg_kernel(ei[1])
    h1 = _mm1(x, W1)
    dinv = _dinv(hist)
    h1p = _scale(h1, dinv)
    p1 = _scatter_kernel(_pad_rows(h1p), src, dst, zeros)
    h2p = _mid(p1, h1p, dinv, b1.reshape(1, NFEAT), W2)
    p2 = _scatter_kernel(_pad_rows(h2p), src, dst, zeros)
    out = _fin(p2, h2p, dinv, b2.reshape(1, NFEAT), Wfc, bfc.reshape(1, 1))
    return out.reshape(N_NODES)


# speedup vs baseline: 2.6137x; 2.6137x over previous
import functools

import jax
import jax.numpy as jnp
from jax import lax
from jax.experimental import pallas as pl
from jax.experimental.pallas import tpu as pltpu
from jax.experimental.pallas import tpu_sc as plsc

N_NODES = 10000
N_EDGES = 320000
NFEAT = 128

NC = 2
NS = 16
NW = NC * NS
EPT = N_EDGES // NW
CHUNK = 384
NFULL = 27
EPT_PAD = NFULL * CHUNK
N_PAD = 10032
ROWS_PER_TILE = 640

_mesh = plsc.VectorSubcoreMesh(core_axis_name="c", subcore_axis_name="s")


@functools.partial(
    pl.kernel,
    mesh=_mesh,
    out_type=jax.ShapeDtypeStruct((NW, N_NODES), jnp.float32),
    scratch_types=[
        pltpu.VMEM((N_NODES,), jnp.float32),
        pltpu.VMEM((EPT,), jnp.int32),
    ],
    compiler_params=pltpu.CompilerParams(needs_layout_passes=False),
)
def _deg_kernel(dst_hbm, out_hbm, hist, dstv):
    c = lax.axis_index("c")
    s = lax.axis_index("s")
    wid = c * NS + s

    def zero(i, carry):
        hist[pl.ds(i * 16, 16)] = jnp.zeros((16,), jnp.float32)
        return carry

    lax.fori_loop(0, N_NODES // 16, zero, 0)

    pltpu.sync_copy(dst_hbm.at[pl.ds(wid * EPT, EPT)], dstv)
    ones = jnp.ones((16,), jnp.float32)

    def step(i, carry):
        idx = dstv[pl.ds(i * 16, 16)]
        plsc.addupdate_scatter(hist, [idx], ones)
        return carry

    lax.fori_loop(0, EPT // 16, step, 0)
    pltpu.sync_copy(hist, out_hbm.at[wid])


@functools.partial(
    pl.kernel,
    mesh=_mesh,
    out_type=jax.ShapeDtypeStruct((NC, N_PAD, NFEAT), jnp.float32),
    scratch_types=[
        pltpu.VMEM_SHARED((N_PAD, NFEAT), jnp.float32),
        pltpu.VMEM((CHUNK,), jnp.int32),
        pltpu.VMEM((CHUNK,), jnp.int32),
        pltpu.VMEM((CHUNK, NFEAT), jnp.float32),
        pltpu.SemaphoreType.DMA,
    ],
    compiler_params=pltpu.CompilerParams(needs_layout_passes=False),
)
def _scatter_kernel(h_hbm, src_hbm, dst_hbm, zeros_hbm, out_hbm,
                    acc, srcv, dstv, rows, gsem):
    c = lax.axis_index("c")
    s = lax.axis_index("s")

    @pl.when(s < NS - 1)
    def _():
        pltpu.sync_copy(zeros_hbm.at[pl.ds(s * ROWS_PER_TILE, ROWS_PER_TILE)],
                        acc.at[pl.ds(s * ROWS_PER_TILE, ROWS_PER_TILE)])

    @pl.when(s == NS - 1)
    def _():
        last = (NS - 1) * ROWS_PER_TILE
        pltpu.sync_copy(zeros_hbm.at[pl.ds(last, N_PAD - last)],
                        acc.at[pl.ds(last, N_PAD - last)])

    plsc.subcore_barrier()

    wid = c * NS + s
    ebase = wid * EPT_PAD

    def step(i, carry):
        base = ebase + i * CHUNK
        pltpu.sync_copy(src_hbm.at[pl.ds(base, CHUNK)], srcv)
        pltpu.sync_copy(dst_hbm.at[pl.ds(base, CHUNK)], dstv)
        pltpu.async_copy(h_hbm.at[srcv], rows, gsem).wait()
        pltpu.sync_copy(rows, acc.at[dstv], add=True)
        return carry

    lax.fori_loop(0, NFULL, step, 0)
    plsc.subcore_barrier()

    @pl.when(s < NS - 1)
    def _():
        pltpu.sync_copy(acc.at[pl.ds(s * ROWS_PER_TILE, ROWS_PER_TILE)],
                        out_hbm.at[c, pl.ds(s * ROWS_PER_TILE, ROWS_PER_TILE)])

    @pl.when(s == NS - 1)
    def _():
        last = (NS - 1) * ROWS_PER_TILE
        pltpu.sync_copy(acc.at[pl.ds(last, N_PAD - last)],
                        out_hbm.at[c, pl.ds(last, N_PAD - last)])


_RB = 1000
_GRID = N_NODES // _RB


def _mm1_body(x_ref, w_ref, o_ref):
    o_ref[...] = jnp.dot(x_ref[...], w_ref[...],
                         preferred_element_type=jnp.float32)


def _mm1(x, w):
    return pl.pallas_call(
        _mm1_body,
        grid=(_GRID,),
        in_specs=[
            pl.BlockSpec((_RB, NFEAT), lambda i: (i, 0)),
            pl.BlockSpec((NFEAT, NFEAT), lambda i: (0, 0)),
        ],
        out_specs=pl.BlockSpec((_RB, NFEAT), lambda i: (i, 0)),
        out_shape=jax.ShapeDtypeStruct((N_NODES, NFEAT), jnp.float32),
    )(x, w)


def _dinv_body(hist_ref, dinv_ref):
    deg = 1.0 + jnp.sum(hist_ref[...], axis=0)
    dinv_ref[...] = lax.rsqrt(deg)[:, None]


def _dinv(hist):
    return pl.pallas_call(
        _dinv_body,
        grid=(1,),
        in_specs=[pl.BlockSpec((NW, N_NODES), lambda i: (0, 0))],
        out_specs=pl.BlockSpec((N_NODES, 1), lambda i: (0, 0)),
        out_shape=jax.ShapeDtypeStruct((N_NODES, 1), jnp.float32),
    )(hist)


def _scale_body(h1_ref, dinv_ref, h1p_ref):
    h1p_ref[...] = h1_ref[...] * dinv_ref[...]


def _scale(h1, dinv):
    return pl.pallas_call(
        _scale_body,
        grid=(_GRID,),
        in_specs=[
            pl.BlockSpec((_RB, NFEAT), lambda i: (i, 0)),
            pl.BlockSpec((_RB, 1), lambda i: (i, 0)),
        ],
        out_specs=pl.BlockSpec((_RB, NFEAT), lambda i: (i, 0)),
        out_shape=jax.ShapeDtypeStruct((N_NODES, NFEAT), jnp.float32),
    )(h1, dinv)


def _mid_body(p_ref, h1p_ref, dinv_ref, b1_ref, w2_ref, h2p_ref):
    psum = p_ref[0] + p_ref[1]
    u = (psum + h1p_ref[...]) * dinv_ref[...] + b1_ref[...]
    u = jnp.maximum(u, 0.0)
    h2 = jnp.dot(u, w2_ref[...], preferred_element_type=jnp.float32)
    h2p_ref[...] = h2 * dinv_ref[...]


def _mid(p1, h1p, dinv, b1, w2):
    return pl.pallas_call(
        _mid_body,
        grid=(_GRID,),
        in_specs=[
            pl.BlockSpec((NC, _RB, NFEAT), lambda i: (0, i, 0)),
            pl.BlockSpec((_RB, NFEAT), lambda i: (i, 0)),
            pl.BlockSpec((_RB, 1), lambda i: (i, 0)),
            pl.BlockSpec((1, NFEAT), lambda i: (0, 0)),
            pl.BlockSpec((NFEAT, NFEAT), lambda i: (0, 0)),
        ],
        out_specs=pl.BlockSpec((_RB, NFEAT), lambda i: (i, 0)),
        out_shape=jax.ShapeDtypeStruct((N_NODES, NFEAT), jnp.float32),
    )(p1, h1p, dinv, b1, w2)


def _fin_body(p_ref, h2p_ref, dinv_ref, b2_ref, wfc_ref, bfc_ref, o_ref):
    v = (p_ref[0] + p_ref[1] + h2p_ref[...]) * dinv_ref[...] + b2_ref[...]
    o_ref[...] = jnp.dot(v, wfc_ref[...],
                         preferred_element_type=jnp.float32) + bfc_ref[0, 0]


def _fin(p2, h2p, dinv, b2, wfc, bfc):
    return pl.pallas_call(
        _fin_body,
        grid=(_GRID,),
        in_specs=[
            pl.BlockSpec((NC, _RB, NFEAT), lambda i: (0, i, 0)),
            pl.BlockSpec((_RB, NFEAT), lambda i: (i, 0)),
            pl.BlockSpec((_RB, 1), lambda i: (i, 0)),
            pl.BlockSpec((1, NFEAT), lambda i: (0, 0)),
            pl.BlockSpec((NFEAT, 1), lambda i: (0, 0)),
            pl.BlockSpec((1, 1), lambda i: (0, 0)),
        ],
        out_specs=pl.BlockSpec((_RB, 1), lambda i: (i, 0)),
        out_shape=jax.ShapeDtypeStruct((N_NODES, 1), jnp.float32),
    )(p2, h2p, dinv, b2, wfc, bfc)


def _pad_edges(v):
    pad = jnp.broadcast_to(
        N_NODES + jnp.arange(NW, dtype=jnp.int32)[:, None],
        (NW, EPT_PAD - EPT))
    out = jnp.concatenate([v.reshape(NW, EPT), pad], axis=1)
    return out.reshape(-1)


def _pad_rows(h):
    return jnp.concatenate(
        [h, jnp.zeros((N_PAD - N_NODES, NFEAT), jnp.float32)], axis=0)


def kernel(x, edge_index, W1, b1, W2, b2, Wfc, bfc):
    ei = edge_index.astype(jnp.int32)
    src = _pad_edges(ei[0])
    dst = _pad_edges(ei[1])
    zeros = jnp.zeros((N_PAD, NFEAT), jnp.float32)

    hist = _deg_kernel(ei[1])
    h1 = _mm1(x, W1)
    dinv = _dinv(hist)
    h1p = _scale(h1, dinv)
    p1 = _scatter_kernel(_pad_rows(h1p), src, dst, zeros)
    h2p = _mid(p1, h1p, dinv, b1.reshape(1, NFEAT), W2)
    p2 = _scatter_kernel(_pad_rows(h2p), src, dst, zeros)
    out = _fin(p2, h2p, dinv, b2.reshape(1, NFEAT), Wfc, bfc.reshape(1, 1))
    return out.reshape(N_NODES)


# speedup vs baseline: 2.6763x; 1.0240x over previous
import functools

import jax
import jax.numpy as jnp
from jax import lax
from jax.experimental import pallas as pl
from jax.experimental.pallas import tpu as pltpu
from jax.experimental.pallas import tpu_sc as plsc

N_NODES = 10000
N_EDGES = 320000
NFEAT = 128

NC = 2
NS = 16
NW = NC * NS
EPT = N_EDGES // NW
CHUNK = 192
NFULL = 54
EPT_PAD = NFULL * CHUNK
N_PAD = 10032
ROWS_PER_TILE = 640

_mesh = plsc.VectorSubcoreMesh(core_axis_name="c", subcore_axis_name="s")


@functools.partial(
    pl.kernel,
    mesh=_mesh,
    out_type=jax.ShapeDtypeStruct((NW, N_NODES), jnp.float32),
    scratch_types=[
        pltpu.VMEM((N_NODES,), jnp.float32),
        pltpu.VMEM((EPT,), jnp.int32),
    ],
    compiler_params=pltpu.CompilerParams(needs_layout_passes=False),
)
def _deg_kernel(dst_hbm, out_hbm, hist, dstv):
    c = lax.axis_index("c")
    s = lax.axis_index("s")
    wid = c * NS + s

    def zero(i, carry):
        hist[pl.ds(i * 16, 16)] = jnp.zeros((16,), jnp.float32)
        return carry

    lax.fori_loop(0, N_NODES // 16, zero, 0)

    pltpu.sync_copy(dst_hbm.at[pl.ds(wid * EPT, EPT)], dstv)
    ones = jnp.ones((16,), jnp.float32)

    def step(i, carry):
        idx = dstv[pl.ds(i * 16, 16)]
        plsc.addupdate_scatter(hist, [idx], ones)
        return carry

    lax.fori_loop(0, EPT // 16, step, 0)
    pltpu.sync_copy(hist, out_hbm.at[wid])


@functools.partial(
    pl.kernel,
    mesh=_mesh,
    out_type=jax.ShapeDtypeStruct((NC, N_PAD, NFEAT), jnp.float32),
    scratch_types=[
        pltpu.VMEM_SHARED((N_PAD, NFEAT), jnp.float32),
        pltpu.VMEM((CHUNK,), jnp.int32),
        pltpu.VMEM((CHUNK,), jnp.int32),
        pltpu.VMEM((CHUNK,), jnp.int32),
        pltpu.VMEM((CHUNK,), jnp.int32),
        pltpu.VMEM((CHUNK, NFEAT), jnp.float32),
        pltpu.VMEM((CHUNK, NFEAT), jnp.float32),
        pltpu.SemaphoreType.DMA,
        pltpu.SemaphoreType.DMA,
        pltpu.SemaphoreType.DMA,
        pltpu.SemaphoreType.DMA,
    ],
    compiler_params=pltpu.CompilerParams(needs_layout_passes=False),
)
def _scatter_kernel(h_hbm, src_hbm, dst_hbm, zeros_hbm, out_hbm,
                    acc, srcv0, dstv0, srcv1, dstv1, rows0, rows1,
                    gsem0, gsem1, ssem0, ssem1):
    c = lax.axis_index("c")
    s = lax.axis_index("s")

    @pl.when(s < NS - 1)
    def _():
        pltpu.sync_copy(zeros_hbm.at[pl.ds(s * ROWS_PER_TILE, ROWS_PER_TILE)],
                        acc.at[pl.ds(s * ROWS_PER_TILE, ROWS_PER_TILE)])

    @pl.when(s == NS - 1)
    def _():
        last = (NS - 1) * ROWS_PER_TILE
        pltpu.sync_copy(zeros_hbm.at[pl.ds(last, N_PAD - last)],
                        acc.at[pl.ds(last, N_PAD - last)])

    plsc.subcore_barrier()

    wid = c * NS + s
    ebase = wid * EPT_PAD

    def slot_step(i, srcv, dstv, rows, gsem, ssem, drain):
        if drain:
            pltpu.make_async_copy(rows, acc.at[dstv], ssem).wait()
        base = ebase + i * CHUNK
        pltpu.sync_copy(src_hbm.at[pl.ds(base, CHUNK)], srcv)
        pltpu.sync_copy(dst_hbm.at[pl.ds(base, CHUNK)], dstv)
        pltpu.async_copy(h_hbm.at[srcv], rows, gsem).wait()
        pltpu.async_copy(rows, acc.at[dstv], ssem, add=True)

    slot_step(0, srcv0, dstv0, rows0, gsem0, ssem0, drain=False)
    slot_step(1, srcv1, dstv1, rows1, gsem1, ssem1, drain=False)

    def pair(k, carry):
        slot_step(2 * k, srcv0, dstv0, rows0, gsem0, ssem0, drain=True)
        slot_step(2 * k + 1, srcv1, dstv1, rows1, gsem1, ssem1, drain=True)
        return carry

    lax.fori_loop(1, NFULL // 2, pair, 0)
    pltpu.make_async_copy(rows0, acc.at[dstv0], ssem0).wait()
    pltpu.make_async_copy(rows1, acc.at[dstv1], ssem1).wait()
    plsc.subcore_barrier()

    @pl.when(s < NS - 1)
    def _():
        pltpu.sync_copy(acc.at[pl.ds(s * ROWS_PER_TILE, ROWS_PER_TILE)],
                        out_hbm.at[c, pl.ds(s * ROWS_PER_TILE, ROWS_PER_TILE)])

    @pl.when(s == NS - 1)
    def _():
        last = (NS - 1) * ROWS_PER_TILE
        pltpu.sync_copy(acc.at[pl.ds(last, N_PAD - last)],
                        out_hbm.at[c, pl.ds(last, N_PAD - last)])


_RB = 1000
_GRID = N_NODES // _RB


def _mm1_body(x_ref, w_ref, o_ref):
    o_ref[...] = jnp.dot(x_ref[...], w_ref[...],
                         preferred_element_type=jnp.float32)


def _mm1(x, w):
    return pl.pallas_call(
        _mm1_body,
        grid=(_GRID,),
        in_specs=[
            pl.BlockSpec((_RB, NFEAT), lambda i: (i, 0)),
            pl.BlockSpec((NFEAT, NFEAT), lambda i: (0, 0)),
        ],
        out_specs=pl.BlockSpec((_RB, NFEAT), lambda i: (i, 0)),
        out_shape=jax.ShapeDtypeStruct((N_NODES, NFEAT), jnp.float32),
    )(x, w)


def _dinv_body(hist_ref, dinv_ref):
    deg = 1.0 + jnp.sum(hist_ref[...], axis=0)
    dinv_ref[...] = lax.rsqrt(deg)[:, None]


def _dinv(hist):
    return pl.pallas_call(
        _dinv_body,
        grid=(1,),
        in_specs=[pl.BlockSpec((NW, N_NODES), lambda i: (0, 0))],
        out_specs=pl.BlockSpec((N_NODES, 1), lambda i: (0, 0)),
        out_shape=jax.ShapeDtypeStruct((N_NODES, 1), jnp.float32),
    )(hist)


def _scale_body(h1_ref, dinv_ref, h1p_ref):
    h1p_ref[...] = h1_ref[...] * dinv_ref[...]


def _scale(h1, dinv):
    return pl.pallas_call(
        _scale_body,
        grid=(_GRID,),
        in_specs=[
            pl.BlockSpec((_RB, NFEAT), lambda i: (i, 0)),
            pl.BlockSpec((_RB, 1), lambda i: (i, 0)),
        ],
        out_specs=pl.BlockSpec((_RB, NFEAT), lambda i: (i, 0)),
        out_shape=jax.ShapeDtypeStruct((N_NODES, NFEAT), jnp.float32),
    )(h1, dinv)


def _mid_body(p_ref, h1p_ref, dinv_ref, b1_ref, w2_ref, h2p_ref):
    psum = p_ref[0] + p_ref[1]
    u = (psum + h1p_ref[...]) * dinv_ref[...] + b1_ref[...]
    u = jnp.maximum(u, 0.0)
    h2 = jnp.dot(u, w2_ref[...], preferred_element_type=jnp.float32)
    h2p_ref[...] = h2 * dinv_ref[...]


def _mid(p1, h1p, dinv, b1, w2):
    return pl.pallas_call(
        _mid_body,
        grid=(_GRID,),
        in_specs=[
            pl.BlockSpec((NC, _RB, NFEAT), lambda i: (0, i, 0)),
            pl.BlockSpec((_RB, NFEAT), lambda i: (i, 0)),
            pl.BlockSpec((_RB, 1), lambda i: (i, 0)),
            pl.BlockSpec((1, NFEAT), lambda i: (0, 0)),
            pl.BlockSpec((NFEAT, NFEAT), lambda i: (0, 0)),
        ],
        out_specs=pl.BlockSpec((_RB, NFEAT), lambda i: (i, 0)),
        out_shape=jax.ShapeDtypeStruct((N_NODES, NFEAT), jnp.float32),
    )(p1, h1p, dinv, b1, w2)


def _fin_body(p_ref, h2p_ref, dinv_ref, b2_ref, wfc_ref, bfc_ref, o_ref):
    v = (p_ref[0] + p_ref[1] + h2p_ref[...]) * dinv_ref[...] + b2_ref[...]
    o_ref[...] = jnp.dot(v, wfc_ref[...],
                         preferred_element_type=jnp.float32) + bfc_ref[0, 0]


def _fin(p2, h2p, dinv, b2, wfc, bfc):
    return pl.pallas_call(
        _fin_body,
        grid=(_GRID,),
        in_specs=[
            pl.BlockSpec((NC, _RB, NFEAT), lambda i: (0, i, 0)),
            pl.BlockSpec((_RB, NFEAT), lambda i: (i, 0)),
            pl.BlockSpec((_RB, 1), lambda i: (i, 0)),
            pl.BlockSpec((1, NFEAT), lambda i: (0, 0)),
            pl.BlockSpec((NFEAT, 1), lambda i: (0, 0)),
            pl.BlockSpec((1, 1), lambda i: (0, 0)),
        ],
        out_specs=pl.BlockSpec((_RB, 1), lambda i: (i, 0)),
        out_shape=jax.ShapeDtypeStruct((N_NODES, 1), jnp.float32),
    )(p2, h2p, dinv, b2, wfc, bfc)


def _pad_edges(v):
    pad = jnp.broadcast_to(
        N_NODES + jnp.arange(NW, dtype=jnp.int32)[:, None],
        (NW, EPT_PAD - EPT))
    out = jnp.concatenate([v.reshape(NW, EPT), pad], axis=1)
    return out.reshape(-1)


def _pad_rows(h):
    return jnp.concatenate(
        [h, jnp.zeros((N_PAD - N_NODES, NFEAT), jnp.float32)], axis=0)


def kernel(x, edge_index, W1, b1, W2, b2, Wfc, bfc):
    ei = edge_index.astype(jnp.int32)
    src = _pad_edges(ei[0])
    dst = _pad_edges(ei[1])
    zeros = jnp.zeros((N_PAD, NFEAT), jnp.float32)

    hist = _deg_kernel(ei[1])
    h1 = _mm1(x, W1)
    dinv = _dinv(hist)
    h1p = _scale(h1, dinv)
    p1 = _scatter_kernel(_pad_rows(h1p), src, dst, zeros)
    h2p = _mid(p1, h1p, dinv, b1.reshape(1, NFEAT), W2)
    p2 = _scatter_kernel(_pad_rows(h2p), src, dst, zeros)
    out = _fin(p2, h2p, dinv, b2.reshape(1, NFEAT), Wfc, bfc.reshape(1, 1))
    return out.reshape(N_NODES)


# speedup vs baseline: 2.7417x; 1.0244x over previous
import functools

import jax
import jax.numpy as jnp
from jax import lax
from jax.experimental import pallas as pl
from jax.experimental.pallas import tpu as pltpu
from jax.experimental.pallas import tpu_sc as plsc

N_NODES = 10000
N_EDGES = 320000
NFEAT = 128

NC = 2
NS = 16
NW = NC * NS
EPT = N_EDGES // NW
CHUNK = 368
NFULL = 28
EPT_PAD = NFULL * CHUNK
N_PAD = 10240
RPT = N_PAD // NS

_mesh = plsc.VectorSubcoreMesh(core_axis_name="c", subcore_axis_name="s")


@functools.partial(
    pl.kernel,
    mesh=_mesh,
    out_type=jax.ShapeDtypeStruct((NW, N_PAD), jnp.float32),
    scratch_types=[
        pltpu.VMEM((N_PAD,), jnp.float32),
        pltpu.VMEM((EPT,), jnp.int32),
    ],
    compiler_params=pltpu.CompilerParams(needs_layout_passes=False),
)
def _deg_kernel(dst_hbm, out_hbm, hist, dstv):
    c = lax.axis_index("c")
    s = lax.axis_index("s")
    wid = c * NS + s

    def zero(i, carry):
        hist[pl.ds(i * 16, 16)] = jnp.zeros((16,), jnp.float32)
        return carry

    lax.fori_loop(0, N_PAD // 16, zero, 0)

    pltpu.sync_copy(dst_hbm.at[pl.ds(wid * EPT, EPT)], dstv)
    ones = jnp.ones((16,), jnp.float32)

    def step(i, carry):
        idx = dstv[pl.ds(i * 16, 16)]
        plsc.addupdate_scatter(hist, [idx], ones)
        return carry

    lax.fori_loop(0, EPT // 16, step, 0)
    pltpu.sync_copy(hist, out_hbm.at[wid])


@functools.partial(
    pl.kernel,
    mesh=_mesh,
    out_type=jax.ShapeDtypeStruct((NC, N_PAD, NFEAT), jnp.float32),
    scratch_types=[
        pltpu.VMEM_SHARED((N_PAD, NFEAT), jnp.float32),
        pltpu.VMEM((CHUNK,), jnp.int32),
        pltpu.VMEM((CHUNK,), jnp.int32),
        pltpu.VMEM((CHUNK, NFEAT), jnp.float32),
        pltpu.SemaphoreType.DMA,
    ],
    compiler_params=pltpu.CompilerParams(needs_layout_passes=False),
)
def _scatter_kernel(h_hbm, src_hbm, dst_hbm, zeros_hbm, out_hbm,
                    acc, srcv, dstv, rows, gsem):
    c = lax.axis_index("c")
    s = lax.axis_index("s")

    pltpu.sync_copy(zeros_hbm.at[pl.ds(s * RPT, RPT)],
                    acc.at[pl.ds(s * RPT, RPT)])
    plsc.subcore_barrier()

    wid = c * NS + s
    ebase = wid * EPT_PAD

    def step(i, carry):
        base = ebase + i * CHUNK
        pltpu.sync_copy(src_hbm.at[pl.ds(base, CHUNK)], srcv)
        pltpu.sync_copy(dst_hbm.at[pl.ds(base, CHUNK)], dstv)
        pltpu.async_copy(h_hbm.at[srcv], rows, gsem).wait()
        pltpu.sync_copy(rows, acc.at[dstv], add=True)
        return carry

    lax.fori_loop(0, NFULL, step, 0)
    plsc.subcore_barrier()

    pltpu.sync_copy(acc.at[pl.ds(s * RPT, RPT)],
                    out_hbm.at[c, pl.ds(s * RPT, RPT)])


_RB = 1024
_GRID = N_PAD // _RB


def _in_body(hist_ref, x_ref, w_ref, h1p_ref, dinv_ref):
    deg = 1.0 + jnp.sum(hist_ref[...], axis=0)
    dinv = lax.rsqrt(deg)[:, None]
    h1 = jnp.dot(x_ref[...], w_ref[...], preferred_element_type=jnp.float32)
    h1p_ref[...] = h1 * dinv
    dinv_ref[...] = dinv


def _gcn_in(hist, x, w1):
    return pl.pallas_call(
        _in_body,
        grid=(_GRID,),
        in_specs=[
            pl.BlockSpec((NW, _RB), lambda i: (0, i)),
            pl.BlockSpec((_RB, NFEAT), lambda i: (i, 0)),
            pl.BlockSpec((NFEAT, NFEAT), lambda i: (0, 0)),
        ],
        out_specs=[
            pl.BlockSpec((_RB, NFEAT), lambda i: (i, 0)),
            pl.BlockSpec((_RB, 1), lambda i: (i, 0)),
        ],
        out_shape=[
            jax.ShapeDtypeStruct((N_PAD, NFEAT), jnp.float32),
            jax.ShapeDtypeStruct((N_PAD, 1), jnp.float32),
        ],
    )(hist, x, w1)


def _mid_body(p_ref, h1p_ref, dinv_ref, b1_ref, w2_ref, h2p_ref):
    psum = p_ref[0] + p_ref[1]
    u = (psum + h1p_ref[...]) * dinv_ref[...] + b1_ref[...]
    u = jnp.maximum(u, 0.0)
    h2 = jnp.dot(u, w2_ref[...], preferred_element_type=jnp.float32)
    h2p_ref[...] = h2 * dinv_ref[...]


def _mid(p1, h1p, dinv, b1, w2):
    return pl.pallas_call(
        _mid_body,
        grid=(_GRID,),
        in_specs=[
            pl.BlockSpec((NC, _RB, NFEAT), lambda i: (0, i, 0)),
            pl.BlockSpec((_RB, NFEAT), lambda i: (i, 0)),
            pl.BlockSpec((_RB, 1), lambda i: (i, 0)),
            pl.BlockSpec((1, NFEAT), lambda i: (0, 0)),
            pl.BlockSpec((NFEAT, NFEAT), lambda i: (0, 0)),
        ],
        out_specs=pl.BlockSpec((_RB, NFEAT), lambda i: (i, 0)),
        out_shape=jax.ShapeDtypeStruct((N_PAD, NFEAT), jnp.float32),
    )(p1, h1p, dinv, b1, w2)


def _fin_body(p_ref, h2p_ref, dinv_ref, b2_ref, wfc_ref, bfc_ref, o_ref):
    v = (p_ref[0] + p_ref[1] + h2p_ref[...]) * dinv_ref[...] + b2_ref[...]
    o_ref[...] = jnp.dot(v, wfc_ref[...],
                         preferred_element_type=jnp.float32) + bfc_ref[0, 0]


def _fin(p2, h2p, dinv, b2, wfc, bfc):
    return pl.pallas_call(
        _fin_body,
        grid=(_GRID,),
        in_specs=[
            pl.BlockSpec((NC, _RB, NFEAT), lambda i: (0, i, 0)),
            pl.BlockSpec((_RB, NFEAT), lambda i: (i, 0)),
            pl.BlockSpec((_RB, 1), lambda i: (i, 0)),
            pl.BlockSpec((1, NFEAT), lambda i: (0, 0)),
            pl.BlockSpec((NFEAT, 1), lambda i: (0, 0)),
            pl.BlockSpec((1, 1), lambda i: (0, 0)),
        ],
        out_specs=pl.BlockSpec((_RB, 1), lambda i: (i, 0)),
        out_shape=jax.ShapeDtypeStruct((N_PAD, 1), jnp.float32),
    )(p2, h2p, dinv, b2, wfc, bfc)


def _pad_edges(v):
    pad = jnp.broadcast_to(
        N_NODES + jnp.arange(NW, dtype=jnp.int32)[:, None],
        (NW, EPT_PAD - EPT))
    out = jnp.concatenate([v.reshape(NW, EPT), pad], axis=1)
    return out.reshape(-1)


def kernel(x, edge_index, W1, b1, W2, b2, Wfc, bfc):
    ei = edge_index.astype(jnp.int32)
    src = _pad_edges(ei[0])
    dst = _pad_edges(ei[1])
    zeros = jnp.zeros((N_PAD, NFEAT), jnp.float32)
    x_pad = jnp.concatenate(
        [x, jnp.zeros((N_PAD - N_NODES, NFEAT), jnp.float32)], axis=0)

    hist = _deg_kernel(ei[1])
    h1p, dinv = _gcn_in(hist, x_pad, W1)
    p1 = _scatter_kernel(h1p, src, dst, zeros)
    h2p = _mid(p1, h1p, dinv, b1.reshape(1, NFEAT), W2)
    p2 = _scatter_kernel(h2p, src, dst, zeros)
    out = _fin(p2, h2p, dinv, b2.reshape(1, NFEAT), Wfc, bfc.reshape(1, 1))
    return out.reshape(N_PAD)[:N_NODES]


# speedup vs baseline: 3.0845x; 1.1250x over previous
import functools

import jax
import jax.numpy as jnp
from jax import lax
from jax.experimental import pallas as pl
from jax.experimental.pallas import tpu as pltpu
from jax.experimental.pallas import tpu_sc as plsc

N_NODES = 10000
N_EDGES = 320000
NFEAT = 128

NC = 2
NS = 16
NW = NC * NS
EPT = N_EDGES // NW
CHUNK = 368
NFULL = 28
EPT_PAD = NFULL * CHUNK
N_PAD = 10240
RPT = N_PAD // NS

_mesh = plsc.VectorSubcoreMesh(core_axis_name="c", subcore_axis_name="s")


@functools.partial(
    pl.kernel,
    mesh=_mesh,
    out_type=jax.ShapeDtypeStruct((NW, N_PAD), jnp.float32),
    scratch_types=[
        pltpu.VMEM((N_PAD,), jnp.float32),
        pltpu.VMEM((EPT,), jnp.int32),
    ],
    compiler_params=pltpu.CompilerParams(needs_layout_passes=False),
)
def _deg_kernel(dst_hbm, out_hbm, hist, dstv):
    c = lax.axis_index("c")
    s = lax.axis_index("s")
    wid = c * NS + s

    def zero(i, carry):
        hist[pl.ds(i * 16, 16)] = jnp.zeros((16,), jnp.float32)
        return carry

    lax.fori_loop(0, N_PAD // 16, zero, 0)

    pltpu.sync_copy(dst_hbm.at[pl.ds(wid * EPT, EPT)], dstv)
    ones = jnp.ones((16,), jnp.float32)

    def step(i, carry):
        idx = dstv[pl.ds(i * 16, 16)]
        plsc.addupdate_scatter(hist, [idx], ones)
        return carry

    lax.fori_loop(0, EPT // 16, step, 0)
    pltpu.sync_copy(hist, out_hbm.at[wid])


@functools.partial(
    pl.kernel,
    mesh=_mesh,
    out_type=jax.ShapeDtypeStruct((NC, N_PAD, NFEAT), jnp.float32),
    scratch_types=[
        pltpu.VMEM_SHARED((N_PAD, NFEAT), jnp.float32),
        pltpu.VMEM((CHUNK,), jnp.int32),
        pltpu.VMEM((CHUNK,), jnp.int32),
        pltpu.VMEM((CHUNK,), jnp.int32),
        pltpu.VMEM((CHUNK,), jnp.int32),
        pltpu.VMEM((CHUNK, NFEAT), jnp.float32),
        pltpu.SemaphoreType.DMA,
        pltpu.SemaphoreType.DMA,
        pltpu.SemaphoreType.DMA,
        pltpu.SemaphoreType.DMA,
        pltpu.SemaphoreType.DMA,
    ],
    compiler_params=pltpu.CompilerParams(needs_layout_passes=False),
)
def _scatter_kernel(h_hbm, src_hbm, dst_hbm, zeros_hbm, out_hbm,
                    acc, srcv0, dstv0, srcv1, dstv1, rows,
                    gsem, ssem0, dsem0, ssem1, dsem1):
    c = lax.axis_index("c")
    s = lax.axis_index("s")

    pltpu.sync_copy(zeros_hbm.at[pl.ds(s * RPT, RPT)],
                    acc.at[pl.ds(s * RPT, RPT)])
    plsc.subcore_barrier()

    wid = c * NS + s
    ebase = wid * EPT_PAD

    def prefetch(i, srcv, dstv, ssem, dsem):
        base = ebase + i * CHUNK
        pltpu.async_copy(src_hbm.at[pl.ds(base, CHUNK)], srcv, ssem)
        pltpu.async_copy(dst_hbm.at[pl.ds(base, CHUNK)], dstv, dsem)

    def body(i, srcv, dstv, ssem, dsem, nsrcv, ndstv, nssem, ndsem):
        base = ebase + i * CHUNK
        pltpu.make_async_copy(src_hbm.at[pl.ds(base, CHUNK)], srcv,
                              ssem).wait()
        pltpu.make_async_copy(dst_hbm.at[pl.ds(base, CHUNK)], dstv,
                              dsem).wait()
        pltpu.async_copy(h_hbm.at[srcv], rows, gsem).wait()

        @pl.when(i + 1 < NFULL)
        def _():
            prefetch(i + 1, nsrcv, ndstv, nssem, ndsem)

        pltpu.sync_copy(rows, acc.at[dstv], add=True)

    prefetch(0, srcv0, dstv0, ssem0, dsem0)

    def pair(k, carry):
        body(2 * k, srcv0, dstv0, ssem0, dsem0, srcv1, dstv1, ssem1, dsem1)
        body(2 * k + 1, srcv1, dstv1, ssem1, dsem1, srcv0, dstv0, ssem0,
             dsem0)
        return carry

    lax.fori_loop(0, NFULL // 2, pair, 0)
    plsc.subcore_barrier()

    pltpu.sync_copy(acc.at[pl.ds(s * RPT, RPT)],
                    out_hbm.at[c, pl.ds(s * RPT, RPT)])


_RB = 1024
_GRID = N_PAD // _RB


def _in_body(hist_ref, x_ref, w_ref, h1p_ref, dinv_ref):
    deg = 1.0 + jnp.sum(hist_ref[...], axis=0)
    dinv = lax.rsqrt(deg)[:, None]
    h1 = jnp.dot(x_ref[...], w_ref[...], preferred_element_type=jnp.float32)
    h1p_ref[...] = h1 * dinv
    dinv_ref[...] = dinv


def _gcn_in(hist, x, w1):
    return pl.pallas_call(
        _in_body,
        grid=(_GRID,),
        in_specs=[
            pl.BlockSpec((NW, _RB), lambda i: (0, i)),
            pl.BlockSpec((_RB, NFEAT), lambda i: (i, 0)),
            pl.BlockSpec((NFEAT, NFEAT), lambda i: (0, 0)),
        ],
        out_specs=[
            pl.BlockSpec((_RB, NFEAT), lambda i: (i, 0)),
            pl.BlockSpec((_RB, 1), lambda i: (i, 0)),
        ],
        out_shape=[
            jax.ShapeDtypeStruct((N_PAD, NFEAT), jnp.float32),
            jax.ShapeDtypeStruct((N_PAD, 1), jnp.float32),
        ],
    )(hist, x, w1)


def _mid_body(p_ref, h1p_ref, dinv_ref, b1_ref, w2_ref, h2p_ref):
    psum = p_ref[0] + p_ref[1]
    u = (psum + h1p_ref[...]) * dinv_ref[...] + b1_ref[...]
    u = jnp.maximum(u, 0.0)
    h2 = jnp.dot(u, w2_ref[...], preferred_element_type=jnp.float32)
    h2p_ref[...] = h2 * dinv_ref[...]


def _mid(p1, h1p, dinv, b1, w2):
    return pl.pallas_call(
        _mid_body,
        grid=(_GRID,),
        in_specs=[
            pl.BlockSpec((NC, _RB, NFEAT), lambda i: (0, i, 0)),
            pl.BlockSpec((_RB, NFEAT), lambda i: (i, 0)),
            pl.BlockSpec((_RB, 1), lambda i: (i, 0)),
            pl.BlockSpec((1, NFEAT), lambda i: (0, 0)),
            pl.BlockSpec((NFEAT, NFEAT), lambda i: (0, 0)),
        ],
        out_specs=pl.BlockSpec((_RB, NFEAT), lambda i: (i, 0)),
        out_shape=jax.ShapeDtypeStruct((N_PAD, NFEAT), jnp.float32),
    )(p1, h1p, dinv, b1, w2)


def _fin_body(p_ref, h2p_ref, dinv_ref, b2_ref, wfc_ref, bfc_ref, o_ref):
    v = (p_ref[0] + p_ref[1] + h2p_ref[...]) * dinv_ref[...] + b2_ref[...]
    o_ref[...] = jnp.dot(v, wfc_ref[...],
                         preferred_element_type=jnp.float32) + bfc_ref[0, 0]


def _fin(p2, h2p, dinv, b2, wfc, bfc):
    return pl.pallas_call(
        _fin_body,
        grid=(_GRID,),
        in_specs=[
            pl.BlockSpec((NC, _RB, NFEAT), lambda i: (0, i, 0)),
            pl.BlockSpec((_RB, NFEAT), lambda i: (i, 0)),
            pl.BlockSpec((_RB, 1), lambda i: (i, 0)),
            pl.BlockSpec((1, NFEAT), lambda i: (0, 0)),
            pl.BlockSpec((NFEAT, 1), lambda i: (0, 0)),
            pl.BlockSpec((1, 1), lambda i: (0, 0)),
        ],
        out_specs=pl.BlockSpec((_RB, 1), lambda i: (i, 0)),
        out_shape=jax.ShapeDtypeStruct((N_PAD, 1), jnp.float32),
    )(p2, h2p, dinv, b2, wfc, bfc)


def _pad_edges(v):
    pad = jnp.broadcast_to(
        N_NODES + jnp.arange(NW, dtype=jnp.int32)[:, None],
        (NW, EPT_PAD - EPT))
    out = jnp.concatenate([v.reshape(NW, EPT), pad], axis=1)
    return out.reshape(-1)


def kernel(x, edge_index, W1, b1, W2, b2, Wfc, bfc):
    ei = edge_index.astype(jnp.int32)
    src = _pad_edges(ei[0])
    dst = _pad_edges(ei[1])
    zeros = jnp.zeros((N_PAD, NFEAT), jnp.float32)
    x_pad = jnp.concatenate(
        [x, jnp.zeros((N_PAD - N_NODES, NFEAT), jnp.float32)], axis=0)

    hist = _deg_kernel(ei[1])
    h1p, dinv = _gcn_in(hist, x_pad, W1)
    p1 = _scatter_kernel(h1p, src, dst, zeros)
    h2p = _mid(p1, h1p, dinv, b1.reshape(1, NFEAT), W2)
    p2 = _scatter_kernel(h2p, src, dst, zeros)
    out = _fin(p2, h2p, dinv, b2.reshape(1, NFEAT), Wfc, bfc.reshape(1, 1))
    return out.reshape(N_PAD)[:N_NODES]


# speedup vs baseline: 3.5397x; 1.1476x over previous
import functools

import jax
import jax.numpy as jnp
from jax import lax
from jax.experimental import pallas as pl
from jax.experimental.pallas import tpu as pltpu
from jax.experimental.pallas import tpu_sc as plsc

N_NODES = 10000
N_EDGES = 320000
NFEAT = 128

NC = 2
NS = 16
NW = NC * NS
EPT = N_EDGES // NW
CHUNK = 184
NFULL = 56
EPT_PAD = NFULL * CHUNK
N_PAD = 10240
RPT = N_PAD // NS

_mesh = plsc.VectorSubcoreMesh(core_axis_name="c", subcore_axis_name="s")


@functools.partial(
    pl.kernel,
    mesh=_mesh,
    out_type=jax.ShapeDtypeStruct((NW, N_PAD), jnp.float32),
    scratch_types=[
        pltpu.VMEM((N_PAD,), jnp.float32),
        pltpu.VMEM((EPT,), jnp.int32),
    ],
    compiler_params=pltpu.CompilerParams(needs_layout_passes=False),
)
def _deg_kernel(dst_hbm, out_hbm, hist, dstv):
    c = lax.axis_index("c")
    s = lax.axis_index("s")
    wid = c * NS + s

    def zero(i, carry):
        hist[pl.ds(i * 16, 16)] = jnp.zeros((16,), jnp.float32)
        return carry

    lax.fori_loop(0, N_PAD // 16, zero, 0)

    pltpu.sync_copy(dst_hbm.at[pl.ds(wid * EPT, EPT)], dstv)
    ones = jnp.ones((16,), jnp.float32)

    def step(i, carry):
        idx = dstv[pl.ds(i * 16, 16)]
        plsc.addupdate_scatter(hist, [idx], ones)
        return carry

    lax.fori_loop(0, EPT // 16, step, 0)
    pltpu.sync_copy(hist, out_hbm.at[wid])


@functools.partial(
    pl.kernel,
    mesh=_mesh,
    out_type=jax.ShapeDtypeStruct((NC, N_PAD, NFEAT), jnp.float32),
    scratch_types=[
        pltpu.VMEM_SHARED((N_PAD, NFEAT), jnp.float32),
        [pltpu.VMEM((CHUNK,), jnp.int32) for _ in range(4)],
        [pltpu.VMEM((CHUNK,), jnp.int32) for _ in range(4)],
        [pltpu.VMEM((CHUNK, NFEAT), jnp.float32) for _ in range(2)],
        [pltpu.SemaphoreType.DMA for _ in range(4)],
        [pltpu.SemaphoreType.DMA for _ in range(4)],
        [pltpu.SemaphoreType.DMA for _ in range(2)],
        [pltpu.SemaphoreType.DMA for _ in range(2)],
    ],
    compiler_params=pltpu.CompilerParams(needs_layout_passes=False),
)
def _scatter_kernel(h_hbm, src_hbm, dst_hbm, zeros_hbm, out_hbm,
                    acc, srcvs, dstvs, rowss, ssems, dsems, gsems, csems):
    c = lax.axis_index("c")
    s = lax.axis_index("s")

    pltpu.sync_copy(zeros_hbm.at[pl.ds(s * RPT, RPT)],
                    acc.at[pl.ds(s * RPT, RPT)])
    plsc.subcore_barrier()

    wid = c * NS + s
    ebase = wid * EPT_PAD

    def prefetch(i, m):
        base = ebase + i * CHUNK
        pltpu.async_copy(src_hbm.at[pl.ds(base, CHUNK)], srcvs[m], ssems[m])
        pltpu.async_copy(dst_hbm.at[pl.ds(base, CHUNK)], dstvs[m], dsems[m])

    def body(i, m, r, drain):
        base = ebase + i * CHUNK
        pltpu.make_async_copy(src_hbm.at[pl.ds(base, CHUNK)], srcvs[m],
                              ssems[m]).wait()
        pltpu.make_async_copy(dst_hbm.at[pl.ds(base, CHUNK)], dstvs[m],
                              dsems[m]).wait()
        m2 = (m + 2) % 4
        if drain:
            pltpu.make_async_copy(rowss[r], acc.at[dstvs[m2]],
                                  csems[r]).wait()

        @pl.when(i + 2 < NFULL)
        def _():
            prefetch(i + 2, m2)

        pltpu.async_copy(h_hbm.at[srcvs[m]], rowss[r], gsems[r]).wait()
        pltpu.async_copy(rowss[r], acc.at[dstvs[m]], csems[r], add=True)

    prefetch(0, 0)
    prefetch(1, 1)
    body(0, 0, 0, False)
    body(1, 1, 1, False)
    body(2, 2, 0, True)
    body(3, 3, 1, True)

    def quad(k, carry):
        i0 = 4 * k
        body(i0, 0, 0, True)
        body(i0 + 1, 1, 1, True)
        body(i0 + 2, 2, 0, True)
        body(i0 + 3, 3, 1, True)
        return carry

    lax.fori_loop(1, NFULL // 4, quad, 0)
    pltpu.make_async_copy(rowss[0], acc.at[dstvs[2]], csems[0]).wait()
    pltpu.make_async_copy(rowss[1], acc.at[dstvs[3]], csems[1]).wait()
    plsc.subcore_barrier()

    pltpu.sync_copy(acc.at[pl.ds(s * RPT, RPT)],
                    out_hbm.at[c, pl.ds(s * RPT, RPT)])


_RB = 1024
_GRID = N_PAD // _RB


def _in_body(hist_ref, x_ref, w_ref, h1p_ref, dinv_ref):
    deg = 1.0 + jnp.sum(hist_ref[...], axis=0)
    dinv = lax.rsqrt(deg)[:, None]
    h1 = jnp.dot(x_ref[...], w_ref[...], preferred_element_type=jnp.float32)
    h1p_ref[...] = h1 * dinv
    dinv_ref[...] = dinv


def _gcn_in(hist, x, w1):
    return pl.pallas_call(
        _in_body,
        grid=(_GRID,),
        in_specs=[
            pl.BlockSpec((NW, _RB), lambda i: (0, i)),
            pl.BlockSpec((_RB, NFEAT), lambda i: (i, 0)),
            pl.BlockSpec((NFEAT, NFEAT), lambda i: (0, 0)),
        ],
        out_specs=[
            pl.BlockSpec((_RB, NFEAT), lambda i: (i, 0)),
            pl.BlockSpec((_RB, 1), lambda i: (i, 0)),
        ],
        out_shape=[
            jax.ShapeDtypeStruct((N_PAD, NFEAT), jnp.float32),
            jax.ShapeDtypeStruct((N_PAD, 1), jnp.float32),
        ],
    )(hist, x, w1)


def _mid_body(p_ref, h1p_ref, dinv_ref, b1_ref, w2_ref, h2p_ref):
    psum = p_ref[0] + p_ref[1]
    u = (psum + h1p_ref[...]) * dinv_ref[...] + b1_ref[...]
    u = jnp.maximum(u, 0.0)
    h2 = jnp.dot(u, w2_ref[...], preferred_element_type=jnp.float32)
    h2p_ref[...] = h2 * dinv_ref[...]


def _mid(p1, h1p, dinv, b1, w2):
    return pl.pallas_call(
        _mid_body,
        grid=(_GRID,),
        in_specs=[
            pl.BlockSpec((NC, _RB, NFEAT), lambda i: (0, i, 0)),
            pl.BlockSpec((_RB, NFEAT), lambda i: (i, 0)),
            pl.BlockSpec((_RB, 1), lambda i: (i, 0)),
            pl.BlockSpec((1, NFEAT), lambda i: (0, 0)),
            pl.BlockSpec((NFEAT, NFEAT), lambda i: (0, 0)),
        ],
        out_specs=pl.BlockSpec((_RB, NFEAT), lambda i: (i, 0)),
        out_shape=jax.ShapeDtypeStruct((N_PAD, NFEAT), jnp.float32),
    )(p1, h1p, dinv, b1, w2)


def _fin_body(p_ref, h2p_ref, dinv_ref, b2_ref, wfc_ref, bfc_ref, o_ref):
    v = (p_ref[0] + p_ref[1] + h2p_ref[...]) * dinv_ref[...] + b2_ref[...]
    o_ref[...] = jnp.dot(v, wfc_ref[...],
                         preferred_element_type=jnp.float32) + bfc_ref[0, 0]


def _fin(p2, h2p, dinv, b2, wfc, bfc):
    return pl.pallas_call(
        _fin_body,
        grid=(_GRID,),
        in_specs=[
            pl.BlockSpec((NC, _RB, NFEAT), lambda i: (0, i, 0)),
            pl.BlockSpec((_RB, NFEAT), lambda i: (i, 0)),
            pl.BlockSpec((_RB, 1), lambda i: (i, 0)),
            pl.BlockSpec((1, NFEAT), lambda i: (0, 0)),
            pl.BlockSpec((NFEAT, 1), lambda i: (0, 0)),
            pl.BlockSpec((1, 1), lambda i: (0, 0)),
        ],
        out_specs=pl.BlockSpec((_RB, 1), lambda i: (i, 0)),
        out_shape=jax.ShapeDtypeStruct((N_PAD, 1), jnp.float32),
    )(p2, h2p, dinv, b2, wfc, bfc)


def _pad_edges(v):
    pad = jnp.broadcast_to(
        N_NODES + jnp.arange(NW, dtype=jnp.int32)[:, None],
        (NW, EPT_PAD - EPT))
    out = jnp.concatenate([v.reshape(NW, EPT), pad], axis=1)
    return out.reshape(-1)


def kernel(x, edge_index, W1, b1, W2, b2, Wfc, bfc):
    ei = edge_index.astype(jnp.int32)
    src = _pad_edges(ei[0])
    dst = _pad_edges(ei[1])
    zeros = jnp.zeros((N_PAD, NFEAT), jnp.float32)
    x_pad = jnp.concatenate(
        [x, jnp.zeros((N_PAD - N_NODES, NFEAT), jnp.float32)], axis=0)

    hist = _deg_kernel(ei[1])
    h1p, dinv = _gcn_in(hist, x_pad, W1)
    p1 = _scatter_kernel(h1p, src, dst, zeros)
    h2p = _mid(p1, h1p, dinv, b1.reshape(1, NFEAT), W2)
    p2 = _scatter_kernel(h2p, src, dst, zeros)
    out = _fin(p2, h2p, dinv, b2.reshape(1, NFEAT), Wfc, bfc.reshape(1, 1))
    return out.reshape(N_PAD)[:N_NODES]


# speedup vs baseline: 3.5511x; 1.0032x over previous
import functools

import jax
import jax.numpy as jnp
from jax import lax
from jax.experimental import pallas as pl
from jax.experimental.pallas import tpu as pltpu
from jax.experimental.pallas import tpu_sc as plsc

N_NODES = 10000
N_EDGES = 320000
NFEAT = 128

NC = 2
NS = 16
NW = NC * NS
EPT = N_EDGES // NW
CHUNK = 184
NFULL = 56
EPT_PAD = NFULL * CHUNK
N_PAD = 10240
RPT = N_PAD // NS

_mesh = plsc.VectorSubcoreMesh(core_axis_name="c", subcore_axis_name="s")


@functools.partial(
    pl.kernel,
    mesh=_mesh,
    out_type=jax.ShapeDtypeStruct((NW, N_PAD), jnp.float32),
    scratch_types=[
        pltpu.VMEM((N_PAD,), jnp.float32),
        pltpu.VMEM((EPT,), jnp.int32),
    ],
    compiler_params=pltpu.CompilerParams(needs_layout_passes=False),
)
def _deg_kernel(dst_hbm, out_hbm, hist, dstv):
    c = lax.axis_index("c")
    s = lax.axis_index("s")
    wid = c * NS + s

    def zero(i, carry):
        hist[pl.ds(i * 16, 16)] = jnp.zeros((16,), jnp.float32)
        return carry

    lax.fori_loop(0, N_PAD // 16, zero, 0)

    pltpu.sync_copy(dst_hbm.at[pl.ds(wid * EPT, EPT)], dstv)
    ones = jnp.ones((16,), jnp.float32)

    def step(i, carry):
        idx = dstv[pl.ds(i * 16, 16)]
        plsc.addupdate_scatter(hist, [idx], ones)
        return carry

    lax.fori_loop(0, EPT // 16, step, 0)
    pltpu.sync_copy(hist, out_hbm.at[wid])


@functools.partial(
    pl.kernel,
    mesh=_mesh,
    out_type=jax.ShapeDtypeStruct((NC, N_PAD, NFEAT), jnp.float32),
    scratch_types=[
        pltpu.VMEM_SHARED((N_PAD, NFEAT), jnp.float32),
        [pltpu.VMEM((CHUNK,), jnp.int32) for _ in range(4)],
        [pltpu.VMEM((CHUNK,), jnp.int32) for _ in range(4)],
        [pltpu.VMEM((CHUNK, NFEAT), jnp.float32) for _ in range(2)],
        [pltpu.SemaphoreType.DMA for _ in range(4)],
        [pltpu.SemaphoreType.DMA for _ in range(4)],
        [pltpu.SemaphoreType.DMA for _ in range(2)],
        [pltpu.SemaphoreType.DMA for _ in range(2)],
    ],
    compiler_params=pltpu.CompilerParams(needs_layout_passes=False),
)
def _scatter_kernel(h_hbm, src_hbm, dst_hbm, zeros_hbm, out_hbm,
                    acc, srcvs, dstvs, rowss, ssems, dsems, gsems, csems):
    c = lax.axis_index("c")
    s = lax.axis_index("s")
    wid = c * NS + s
    ebase = wid * EPT_PAD

    def prefetch(i, m):
        base = ebase + i * CHUNK
        pltpu.async_copy(src_hbm.at[pl.ds(base, CHUNK)], srcvs[m], ssems[m])
        pltpu.async_copy(dst_hbm.at[pl.ds(base, CHUNK)], dstvs[m], dsems[m])

    def body(i, m, r, drain):
        base = ebase + i * CHUNK
        pltpu.make_async_copy(src_hbm.at[pl.ds(base, CHUNK)], srcvs[m],
                              ssems[m]).wait()
        pltpu.make_async_copy(dst_hbm.at[pl.ds(base, CHUNK)], dstvs[m],
                              dsems[m]).wait()
        m2 = (m + 2) % 4
        if drain:
            pltpu.make_async_copy(rowss[r], acc.at[dstvs[m2]],
                                  csems[r]).wait()

        @pl.when(i + 2 < NFULL)
        def _():
            prefetch(i + 2, m2)

        pltpu.async_copy(h_hbm.at[srcvs[m]], rowss[r], gsems[r]).wait()
        pltpu.async_copy(rowss[r], acc.at[dstvs[m]], csems[r], add=True)

    prefetch(0, 0)
    prefetch(1, 1)

    pltpu.sync_copy(zeros_hbm.at[pl.ds(s * RPT, RPT)],
                    acc.at[pl.ds(s * RPT, RPT)])
    plsc.subcore_barrier()

    body(0, 0, 0, False)
    body(1, 1, 1, False)
    body(2, 2, 0, True)
    body(3, 3, 1, True)

    def quad(k, carry):
        i0 = 4 * k
        body(i0, 0, 0, True)
        body(i0 + 1, 1, 1, True)
        body(i0 + 2, 2, 0, True)
        body(i0 + 3, 3, 1, True)
        return carry

    lax.fori_loop(1, NFULL // 4, quad, 0)
    pltpu.make_async_copy(rowss[0], acc.at[dstvs[2]], csems[0]).wait()
    pltpu.make_async_copy(rowss[1], acc.at[dstvs[3]], csems[1]).wait()
    plsc.subcore_barrier()

    pltpu.sync_copy(acc.at[pl.ds(s * RPT, RPT)],
                    out_hbm.at[c, pl.ds(s * RPT, RPT)])


_RB = 1024
_GRID = N_PAD // _RB


def _in_body(hist_ref, x_ref, w_ref, h1p_ref, dinv_ref):
    deg = 1.0 + jnp.sum(hist_ref[...], axis=0)
    dinv = lax.rsqrt(deg)[:, None]
    h1 = jnp.dot(x_ref[...], w_ref[...], preferred_element_type=jnp.float32)
    h1p_ref[...] = h1 * dinv
    dinv_ref[...] = dinv


def _gcn_in(hist, x, w1):
    return pl.pallas_call(
        _in_body,
        grid=(_GRID,),
        in_specs=[
            pl.BlockSpec((NW, _RB), lambda i: (0, i)),
            pl.BlockSpec((_RB, NFEAT), lambda i: (i, 0)),
            pl.BlockSpec((NFEAT, NFEAT), lambda i: (0, 0)),
        ],
        out_specs=[
            pl.BlockSpec((_RB, NFEAT), lambda i: (i, 0)),
            pl.BlockSpec((_RB, 1), lambda i: (i, 0)),
        ],
        out_shape=[
            jax.ShapeDtypeStruct((N_PAD, NFEAT), jnp.float32),
            jax.ShapeDtypeStruct((N_PAD, 1), jnp.float32),
        ],
    )(hist, x, w1)


def _mid_body(p_ref, h1p_ref, dinv_ref, b1_ref, w2_ref, h2p_ref):
    psum = p_ref[0] + p_ref[1]
    u = (psum + h1p_ref[...]) * dinv_ref[...] + b1_ref[...]
    u = jnp.maximum(u, 0.0)
    h2 = jnp.dot(u, w2_ref[...], preferred_element_type=jnp.float32)
    h2p_ref[...] = h2 * dinv_ref[...]


def _mid(p1, h1p, dinv, b1, w2):
    return pl.pallas_call(
        _mid_body,
        grid=(_GRID,),
        in_specs=[
            pl.BlockSpec((NC, _RB, NFEAT), lambda i: (0, i, 0)),
            pl.BlockSpec((_RB, NFEAT), lambda i: (i, 0)),
            pl.BlockSpec((_RB, 1), lambda i: (i, 0)),
            pl.BlockSpec((1, NFEAT), lambda i: (0, 0)),
            pl.BlockSpec((NFEAT, NFEAT), lambda i: (0, 0)),
        ],
        out_specs=pl.BlockSpec((_RB, NFEAT), lambda i: (i, 0)),
        out_shape=jax.ShapeDtypeStruct((N_PAD, NFEAT), jnp.float32),
    )(p1, h1p, dinv, b1, w2)


def _fin_body(p_ref, h2p_ref, dinv_ref, b2_ref, wfc_ref, bfc_ref, o_ref):
    v = (p_ref[0] + p_ref[1] + h2p_ref[...]) * dinv_ref[...] + b2_ref[...]
    o_ref[...] = jnp.dot(v, wfc_ref[...],
                         preferred_element_type=jnp.float32) + bfc_ref[0, 0]


def _fin(p2, h2p, dinv, b2, wfc, bfc):
    return pl.pallas_call(
        _fin_body,
        grid=(_GRID,),
        in_specs=[
            pl.BlockSpec((NC, _RB, NFEAT), lambda i: (0, i, 0)),
            pl.BlockSpec((_RB, NFEAT), lambda i: (i, 0)),
            pl.BlockSpec((_RB, 1), lambda i: (i, 0)),
            pl.BlockSpec((1, NFEAT), lambda i: (0, 0)),
            pl.BlockSpec((NFEAT, 1), lambda i: (0, 0)),
            pl.BlockSpec((1, 1), lambda i: (0, 0)),
        ],
        out_specs=pl.BlockSpec((_RB, 1), lambda i: (i, 0)),
        out_shape=jax.ShapeDtypeStruct((N_PAD, 1), jnp.float32),
    )(p2, h2p, dinv, b2, wfc, bfc)


def _pad_edges(v):
    pad = jnp.broadcast_to(
        N_NODES + jnp.arange(NW, dtype=jnp.int32)[:, None],
        (NW, EPT_PAD - EPT))
    out = jnp.concatenate([v.reshape(NW, EPT), pad], axis=1)
    return out.reshape(-1)


def kernel(x, edge_index, W1, b1, W2, b2, Wfc, bfc):
    ei = edge_index.astype(jnp.int32)
    src = _pad_edges(ei[0])
    dst = _pad_edges(ei[1])
    zeros = jnp.zeros((N_PAD, NFEAT), jnp.float32)
    x_pad = jnp.concatenate(
        [x, jnp.zeros((N_PAD - N_NODES, NFEAT), jnp.float32)], axis=0)

    hist = _deg_kernel(ei[1])
    h1p, dinv = _gcn_in(hist, x_pad, W1)
    p1 = _scatter_kernel(h1p, src, dst, zeros)
    h2p = _mid(p1, h1p, dinv, b1.reshape(1, NFEAT), W2)
    p2 = _scatter_kernel(h2p, src, dst, zeros)
    out = _fin(p2, h2p, dinv, b2.reshape(1, NFEAT), Wfc, bfc.reshape(1, 1))
    return out.reshape(N_PAD)[:N_NODES]
